# baseline (speedup 1.0000x reference)
import functools

import jax
import jax.numpy as jnp
from jax import lax
from jax.experimental import pallas as pl
from jax.experimental.pallas import tpu as pltpu
from jax.experimental.pallas import tpu_sc as plsc

N_PLAYER = 100000
BATCH = 16384
TEAM_SIZE = 20
T_HALF = TEAM_SIZE // 2

NC = 2
NS = 16
NW = NC * NS
B_PER_W = BATCH // NW
HALF_PER_W = B_PER_W * T_HALF
LANES = 16
GROUPS = B_PER_W // LANES


def _sc_body(team_hbm, skill_hbm, out_hbm,
             table_sh, team_v, list_a, list_b, vals_a, vals_b, out_v,
             sem_t, sem_a, sem_b):
    sid = lax.axis_index("s")
    wid = sid * NC + lax.axis_index("c")
    cp_team = pltpu.async_copy(
        team_hbm.at[:, pl.ds(wid * B_PER_W, B_PER_W)], team_v, sem_t)

    @pl.when(sid == 0)
    def _stage():
        pltpu.sync_copy(skill_hbm, table_sh)

    cp_team.wait()

    def repack_b(i, carry):
        off = i * LANES
        for t in range(T_HALF, TEAM_SIZE):
            list_b[pl.ds((t - T_HALF) * B_PER_W + off, LANES)] = (
                team_v[t, pl.ds(off, LANES)])
        return carry

    lax.fori_loop(0, GROUPS, repack_b, 0)
    cp_b = pltpu.async_copy(skill_hbm.at[list_b], vals_b, sem_b)

    def repack_a(i, carry):
        off = i * LANES
        for t in range(T_HALF):
            list_a[pl.ds(t * B_PER_W + off, LANES)] = team_v[t, pl.ds(off, LANES)]
        return carry

    lax.fori_loop(0, GROUPS, repack_a, 0)
    plsc.subcore_barrier()
    cp_a = pltpu.async_copy(table_sh.at[list_a], vals_a, sem_a)
    cp_a.wait()
    cp_b.wait()

    def group(g, carry):
        off = g * LANES
        acc = vals_a[pl.ds(off, LANES)]
        for t in range(1, T_HALF):
            acc = acc + vals_a[pl.ds(t * B_PER_W + off, LANES)]
        for t in range(T_HALF):
            acc = acc + vals_b[pl.ds(t * B_PER_W + off, LANES)]
        out_v[pl.ds(off, LANES)] = acc
        return carry

    lax.fori_loop(0, GROUPS, group, 0)
    pltpu.sync_copy(out_v, out_hbm.at[pl.ds(wid * B_PER_W, B_PER_W)])


@functools.partial(
    pl.kernel,
    out_type=jax.ShapeDtypeStruct((BATCH,), jnp.float32),
    mesh=plsc.VectorSubcoreMesh(core_axis_name="c", subcore_axis_name="s"),
    compiler_params=pltpu.CompilerParams(needs_layout_passes=False),
    scratch_types=[
        pltpu.VMEM_SHARED((N_PLAYER,), jnp.float32),
        pltpu.VMEM((TEAM_SIZE, B_PER_W), jnp.int32),
        pltpu.VMEM((HALF_PER_W,), jnp.int32),
        pltpu.VMEM((HALF_PER_W,), jnp.int32),
        pltpu.VMEM((HALF_PER_W,), jnp.float32),
        pltpu.VMEM((HALF_PER_W,), jnp.float32),
        pltpu.VMEM((B_PER_W,), jnp.float32),
        pltpu.SemaphoreType.DMA,
        pltpu.SemaphoreType.DMA,
        pltpu.SemaphoreType.DMA,
    ],
)
def _sc_kernel(team_hbm, skill_hbm, out_hbm, *scratch):
    _sc_body(team_hbm, skill_hbm, out_hbm, *scratch)


def kernel(team, skill):
    out = _sc_kernel(team.astype(jnp.int32).T, skill.reshape(-1))
    return out.reshape(BATCH, 1, 1)


# speedup vs baseline: 1.0884x; 1.0884x over previous
import functools

import jax
import jax.numpy as jnp
from jax import lax
from jax.experimental import pallas as pl
from jax.experimental.pallas import tpu as pltpu
from jax.experimental.pallas import tpu_sc as plsc

N_PLAYER = 100000
BATCH = 16384
TEAM_SIZE = 20

NC = 2
NS = 16
NW = NC * NS
B_PER_W = BATCH // NW
IDX_PER_W = B_PER_W * TEAM_SIZE
LANES = 16
GROUPS = B_PER_W // LANES


def _sc_body(team_hbm, skill_hbm, out_hbm,
             table_sh, team_v, list_v, vals_v, out_v, sem_a, sem_b):
    sid = lax.axis_index("s")
    wid = sid * NC + lax.axis_index("c")
    with jax.named_scope("phase_team_dma"):
        cp_team = pltpu.async_copy(
            team_hbm.at[:, pl.ds(wid * B_PER_W, B_PER_W)], team_v, sem_b)

        @pl.when(sid == 0)
        def _stage():
            pltpu.sync_copy(skill_hbm, table_sh)

        plsc.subcore_barrier()
        cp_team.wait()

    with jax.named_scope("phase_repack"):
        def repack(i, carry):
            off = i * LANES
            for t in range(TEAM_SIZE):
                list_v[pl.ds(t * B_PER_W + off, LANES)] = (
                    team_v[t, pl.ds(off, LANES)])
            return carry

        lax.fori_loop(0, GROUPS, repack, 0)

    with jax.named_scope("phase_gather"):
        pltpu.async_copy(table_sh.at[list_v], vals_v, sem_a).wait()

    with jax.named_scope("phase_reduce"):
        def group(g, carry):
            off = g * LANES
            acc = vals_v[pl.ds(off, LANES)]
            for t in range(1, TEAM_SIZE):
                acc = acc + vals_v[pl.ds(t * B_PER_W + off, LANES)]
            out_v[pl.ds(off, LANES)] = acc
            return carry

        lax.fori_loop(0, GROUPS, group, 0)

    with jax.named_scope("phase_out"):
        pltpu.sync_copy(out_v, out_hbm.at[pl.ds(wid * B_PER_W, B_PER_W)])


@functools.partial(
    pl.kernel,
    out_type=jax.ShapeDtypeStruct((BATCH,), jnp.float32),
    mesh=plsc.VectorSubcoreMesh(core_axis_name="c", subcore_axis_name="s"),
    compiler_params=pltpu.CompilerParams(needs_layout_passes=False),
    scratch_types=[
        pltpu.VMEM_SHARED((N_PLAYER,), jnp.float32),
        pltpu.VMEM((TEAM_SIZE, B_PER_W), jnp.int32),
        pltpu.VMEM((IDX_PER_W,), jnp.int32),
        pltpu.VMEM((IDX_PER_W,), jnp.float32),
        pltpu.VMEM((B_PER_W,), jnp.float32),
        pltpu.SemaphoreType.DMA,
        pltpu.SemaphoreType.DMA,
    ],
)
def _sc_kernel(team_hbm, skill_hbm, out_hbm, *scratch):
    _sc_body(team_hbm, skill_hbm, out_hbm, *scratch)


def kernel(team, skill):
    out = _sc_kernel(team.astype(jnp.int32).T, skill.reshape(-1))
    return out.reshape(BATCH, 1, 1)


# speedup vs baseline: 1.1079x; 1.0180x over previous
import functools

import jax
import jax.numpy as jnp
from jax import lax
from jax.experimental import pallas as pl
from jax.experimental.pallas import tpu as pltpu
from jax.experimental.pallas import tpu_sc as plsc

N_PLAYER = 100000
BATCH = 16384
TEAM_SIZE = 20
T_PAD = 24

NC = 2
NS = 16
NW = NC * NS
B_PER_W = BATCH // NW
IDX_PER_W = B_PER_W * TEAM_SIZE
LANES = 16
GROUPS = B_PER_W // LANES


def _sc_body(team_hbm, skill_hbm, out_hbm,
             table_sh, list_v, vals_v, out_v, sem_t, sem_a):
    sid = lax.axis_index("s")
    wid = sid * NC + lax.axis_index("c")
    col = wid * B_PER_W
    copies = [
        pltpu.async_copy(
            team_hbm.at[pl.ds(t * BATCH + col, B_PER_W)],
            list_v.at[pl.ds(t * B_PER_W, B_PER_W)], sem_t)
        for t in range(TEAM_SIZE)
    ]

    @pl.when(sid == 0)
    def _stage():
        pltpu.sync_copy(skill_hbm, table_sh)

    for cp in copies:
        cp.wait()
    plsc.subcore_barrier()
    pltpu.async_copy(table_sh.at[list_v], vals_v, sem_a).wait()

    def group(g, carry):
        off = g * LANES
        acc = vals_v[pl.ds(off, LANES)]
        for t in range(1, TEAM_SIZE):
            acc = acc + vals_v[pl.ds(t * B_PER_W + off, LANES)]
        out_v[pl.ds(off, LANES)] = acc
        return carry

    lax.fori_loop(0, GROUPS, group, 0)
    pltpu.sync_copy(out_v, out_hbm.at[pl.ds(wid * B_PER_W, B_PER_W)])


@functools.partial(
    pl.kernel,
    out_type=jax.ShapeDtypeStruct((BATCH,), jnp.float32),
    mesh=plsc.VectorSubcoreMesh(core_axis_name="c", subcore_axis_name="s"),
    compiler_params=pltpu.CompilerParams(needs_layout_passes=False),
    scratch_types=[
        pltpu.VMEM_SHARED((N_PLAYER,), jnp.float32),
        pltpu.VMEM((IDX_PER_W,), jnp.int32),
        pltpu.VMEM((IDX_PER_W,), jnp.float32),
        pltpu.VMEM((B_PER_W,), jnp.float32),
        pltpu.SemaphoreType.DMA,
        pltpu.SemaphoreType.DMA,
    ],
)
def _sc_kernel(team_hbm, skill_hbm, out_hbm, *scratch):
    _sc_body(team_hbm, skill_hbm, out_hbm, *scratch)


def kernel(team, skill):
    team_flat = jnp.pad(
        team.astype(jnp.int32).T, ((0, T_PAD - TEAM_SIZE), (0, 0))).reshape(-1)
    out = _sc_kernel(team_flat, skill.reshape(-1))
    return out.reshape(BATCH, 1, 1)
